# baseline (speedup 1.0000x reference)
import functools

import jax
import jax.numpy as jnp
from jax import lax
from jax.experimental import pallas as pl
from jax.experimental.pallas import tpu as pltpu
from jax.experimental.pallas import tpu_sc as plsc

N = 10000
D = 128
E = 320000
EPS = 1e-12

NWORKERS = 32
DSL = D // NWORKERS

CHUNK = 16000
NCHUNK = E // CHUNK


def _inv_norm(x2):
    return lax.rsqrt(jnp.maximum(x2, EPS * EPS))


def _eye():
    r = lax.broadcasted_iota(jnp.int32, (D, D), 0)
    c = lax.broadcasted_iota(jnp.int32, (D, D), 1)
    return (r == c).astype(jnp.float32)


def _t_out(m):
    return lax.dot_general(_eye(), m, (((1,), (1,)), ((), ())),
                           precision=lax.Precision.HIGHEST,
                           preferred_element_type=jnp.float32)


def _t_in(mt):
    return lax.dot_general(mt, _eye(), (((0,), (0,)), ((), ())),
                           precision=lax.Precision.HIGHEST,
                           preferred_element_type=jnp.float32)


def _pack_body(ei_ref, pk_ref):
    pk_ref[...] = ei_ref[0:1, :] | (ei_ref[1:2, :] << 16)


_pack_edges = pl.pallas_call(
    _pack_body,
    grid=(10,),
    in_specs=[pl.BlockSpec((2, E // 10), lambda i: (0, i))],
    out_specs=pl.BlockSpec((1, E // 10), lambda i: (0, i)),
    out_shape=jax.ShapeDtypeStruct((1, E), jnp.int32),
)


def _norm_body(x_ref, xnt_ref):
    x = x_ref[...]
    n2 = jnp.sum(x * x, axis=1, keepdims=True)
    xnt_ref[...] = _t_out(x * _inv_norm(n2))


def _mid_body(x_ref, pt_ref, wlts_ref, bl_ref, wrt_ref, h_ref, hnt_ref):
    x = x_ref[...]
    n2 = jnp.sum(x * x, axis=1, keepdims=True)
    xn = x * _inv_norm(n2)
    agg = xn + _t_in(pt_ref[...])
    a2 = jnp.sum(agg * agg, axis=1, keepdims=True)
    mn = agg * (_inv_norm(a2) * jnp.sqrt(n2))
    out = (jnp.dot(mn, wlts_ref[...], preferred_element_type=jnp.float32)
           + bl_ref[...]
           + jnp.dot(x, wrt_ref[...], preferred_element_type=jnp.float32))
    o2 = jnp.sum(out * out, axis=1, keepdims=True)
    h = jnp.maximum(out * _inv_norm(o2), 0.0)
    h_ref[...] = h
    h2 = jnp.sum(h * h, axis=1, keepdims=True)
    hnt_ref[...] = _t_out(h * _inv_norm(h2))


def _final_body(x_ref, pt_ref, wlts_ref, bl_ref, wrt_ref, out_ref):
    x = x_ref[...]
    n2 = jnp.sum(x * x, axis=1, keepdims=True)
    xn = x * _inv_norm(n2)
    agg = xn + _t_in(pt_ref[...])
    a2 = jnp.sum(agg * agg, axis=1, keepdims=True)
    mn = agg * (_inv_norm(a2) * jnp.sqrt(n2))
    out = (jnp.dot(mn, wlts_ref[...], preferred_element_type=jnp.float32)
           + bl_ref[...]
           + jnp.dot(x, wrt_ref[...], preferred_element_type=jnp.float32))
    o2 = jnp.sum(out * out, axis=1, keepdims=True)
    out_ref[...] = out * _inv_norm(o2)


_nat = jax.ShapeDtypeStruct((N, D), jnp.float32)
_tr = jax.ShapeDtypeStruct((D, N), jnp.float32)

_normalize = pl.pallas_call(_norm_body, out_shape=_tr)

_mid = pl.pallas_call(_mid_body, out_shape=[_nat, _tr])

_final = pl.pallas_call(_final_body, out_shape=_nat)


def _scatter_body(xnt_hbm, pk_hbm, out_hbm, xn_t, acc_t, pk_b0, pk_b1, sem0, sem1):
    c = lax.axis_index("c")
    s = lax.axis_index("s")
    wid = s * 2 + c
    d0 = wid * DSL

    first = pltpu.async_copy(pk_hbm.at[0, pl.ds(0, CHUNK)], pk_b0, sem0)
    pltpu.sync_copy(xnt_hbm.at[pl.ds(d0, DSL), :], xn_t)

    zeros = jnp.zeros((16,), jnp.float32)

    @plsc.parallel_loop(0, N // 16, unroll=8)
    def _(g):
        for d in range(DSL):
            acc_t[d, pl.ds(g * 16, 16)] = zeros

    bufs = [pk_b0, pk_b1]
    sems = [sem0, sem1]
    copies = [first, None]
    for ci in range(NCHUNK):
        if ci + 1 < NCHUNK:
            copies[(ci + 1) % 2] = pltpu.async_copy(
                pk_hbm.at[0, pl.ds((ci + 1) * CHUNK, CHUNK)],
                bufs[(ci + 1) % 2], sems[(ci + 1) % 2])
        copies[ci % 2].wait()
        pk_b = bufs[ci % 2]

        @plsc.parallel_loop(0, CHUNK, step=16, unroll=16)
        def _(b):
            pk_v = pk_b[pl.ds(b, 16)]
            src_v = pk_v & 0xFFFF
            dst_v = pk_v >> 16
            for d in range(DSL):
                vals = plsc.load_gather(xn_t.at[d], [src_v])
                plsc.addupdate_scatter(acc_t.at[d], [dst_v], vals)

    pltpu.sync_copy(acc_t, out_hbm.at[pl.ds(d0, DSL), :])


_sc_scatter = functools.partial(
    pl.kernel,
    out_type=_tr,
    mesh=plsc.VectorSubcoreMesh(core_axis_name="c", subcore_axis_name="s"),
    compiler_params=pltpu.CompilerParams(use_tc_tiling_on_sc=False,
                                         needs_layout_passes=False),
    scratch_types=[
        pltpu.VMEM((DSL, N), jnp.float32),
        pltpu.VMEM((DSL, N), jnp.float32),
        pltpu.VMEM((CHUNK,), jnp.int32),
        pltpu.VMEM((CHUNK,), jnp.int32),
        pltpu.SemaphoreType.DMA,
        pltpu.SemaphoreType.DMA,
    ],
)(_scatter_body)


def kernel(x, edge_index, Wl1, bl1, Wr1, scale1, Wl2, bl2, Wr2, scale2):
    wl1ts = (Wl1 * scale1).T
    wl2ts = (Wl2 * scale2).T
    wr1t = Wr1.T
    wr2t = Wr2.T
    bl1r = bl1.reshape(1, D)
    bl2r = bl2.reshape(1, D)

    pk = _pack_edges(edge_index)
    xnt = _normalize(x)
    pt1 = _sc_scatter(xnt, pk)
    h, hnt = _mid(x, pt1, wl1ts, bl1r, wr1t)
    pt2 = _sc_scatter(hnt, pk)
    return _final(h, pt2, wl2ts, bl2r, wr2t)


# speedup vs baseline: 1.0175x; 1.0175x over previous
import functools

import jax
import jax.numpy as jnp
from jax import lax
from jax.experimental import pallas as pl
from jax.experimental.pallas import tpu as pltpu
from jax.experimental.pallas import tpu_sc as plsc

N = 10000
D = 128
E = 320000
EPS = 1e-12

NWORKERS = 32
DSL = D // NWORKERS

CHUNK = 20000
NCHUNK = E // CHUNK


def _inv_norm(x2):
    return lax.rsqrt(jnp.maximum(x2, EPS * EPS))


def _eye():
    r = lax.broadcasted_iota(jnp.int32, (D, D), 0)
    c = lax.broadcasted_iota(jnp.int32, (D, D), 1)
    return (r == c).astype(jnp.float32)


def _t_out(m):
    return lax.dot_general(_eye(), m, (((1,), (1,)), ((), ())),
                           precision=lax.Precision.HIGHEST,
                           preferred_element_type=jnp.float32)


def _t_in(mt):
    return lax.dot_general(mt, _eye(), (((0,), (0,)), ((), ())),
                           precision=lax.Precision.HIGHEST,
                           preferred_element_type=jnp.float32)


def _pack_body(ei_ref, pk_ref):
    pk_ref[...] = ei_ref[0:1, :] | (ei_ref[1:2, :] << 16)


_pack_edges = pl.pallas_call(
    _pack_body,
    grid=(10,),
    in_specs=[pl.BlockSpec((2, E // 10), lambda i: (0, i))],
    out_specs=pl.BlockSpec((1, E // 10), lambda i: (0, i)),
    out_shape=jax.ShapeDtypeStruct((1, E), jnp.int32),
)


def _norm_body(x_ref, xnt_ref):
    x = x_ref[...]
    n2 = jnp.sum(x * x, axis=1, keepdims=True)
    xnt_ref[...] = _t_out(x * _inv_norm(n2))


def _mid_body(x_ref, pt_ref, wlts_ref, bl_ref, wrt_ref, h_ref, hnt_ref):
    x = x_ref[...]
    n2 = jnp.sum(x * x, axis=1, keepdims=True)
    xn = x * _inv_norm(n2)
    agg = xn + _t_in(pt_ref[...])
    a2 = jnp.sum(agg * agg, axis=1, keepdims=True)
    mn = agg * (_inv_norm(a2) * jnp.sqrt(n2))
    out = (jnp.dot(mn, wlts_ref[...], preferred_element_type=jnp.float32)
           + bl_ref[...]
           + jnp.dot(x, wrt_ref[...], preferred_element_type=jnp.float32))
    o2 = jnp.sum(out * out, axis=1, keepdims=True)
    h = jnp.maximum(out * _inv_norm(o2), 0.0)
    h_ref[...] = h
    h2 = jnp.sum(h * h, axis=1, keepdims=True)
    hnt_ref[...] = _t_out(h * _inv_norm(h2))


def _final_body(x_ref, pt_ref, wlts_ref, bl_ref, wrt_ref, out_ref):
    x = x_ref[...]
    n2 = jnp.sum(x * x, axis=1, keepdims=True)
    xn = x * _inv_norm(n2)
    agg = xn + _t_in(pt_ref[...])
    a2 = jnp.sum(agg * agg, axis=1, keepdims=True)
    mn = agg * (_inv_norm(a2) * jnp.sqrt(n2))
    out = (jnp.dot(mn, wlts_ref[...], preferred_element_type=jnp.float32)
           + bl_ref[...]
           + jnp.dot(x, wrt_ref[...], preferred_element_type=jnp.float32))
    o2 = jnp.sum(out * out, axis=1, keepdims=True)
    out_ref[...] = out * _inv_norm(o2)


_nat = jax.ShapeDtypeStruct((N, D), jnp.float32)
_tr = jax.ShapeDtypeStruct((D, N), jnp.float32)

_normalize = pl.pallas_call(_norm_body, out_shape=_tr)

_mid = pl.pallas_call(_mid_body, out_shape=[_nat, _tr])

_final = pl.pallas_call(_final_body, out_shape=_nat)


def _scatter_body(xnt_hbm, pk_hbm, out_hbm, xn_t, acc_t, pk_b0, pk_b1, sem0, sem1):
    c = lax.axis_index("c")
    s = lax.axis_index("s")
    wid = s * 2 + c
    d0 = wid * DSL

    first = pltpu.async_copy(pk_hbm.at[0, pl.ds(0, CHUNK)], pk_b0, sem0)
    stage = pltpu.async_copy(xnt_hbm.at[pl.ds(d0, DSL), :], xn_t, sem1)

    zeros = jnp.zeros((16,), jnp.float32)

    @plsc.parallel_loop(0, N // 16, unroll=8)
    def _(g):
        for d in range(DSL):
            acc_t[d, pl.ds(g * 16, 16)] = zeros

    stage.wait()

    bufs = [pk_b0, pk_b1]
    sems = [sem0, sem1]
    copies = [first, None]
    for ci in range(NCHUNK):
        if ci + 1 < NCHUNK:
            copies[(ci + 1) % 2] = pltpu.async_copy(
                pk_hbm.at[0, pl.ds((ci + 1) * CHUNK, CHUNK)],
                bufs[(ci + 1) % 2], sems[(ci + 1) % 2])
        copies[ci % 2].wait()
        pk_b = bufs[ci % 2]

        @plsc.parallel_loop(0, CHUNK, step=16, unroll=16)
        def _(b):
            pk_v = pk_b[pl.ds(b, 16)]
            src_v = pk_v & 0xFFFF
            dst_v = pk_v >> 16
            for d in range(DSL):
                vals = plsc.load_gather(xn_t.at[d], [src_v])
                plsc.addupdate_scatter(acc_t.at[d], [dst_v], vals)

    pltpu.sync_copy(acc_t, out_hbm.at[pl.ds(d0, DSL), :])


_sc_scatter = functools.partial(
    pl.kernel,
    out_type=_tr,
    mesh=plsc.VectorSubcoreMesh(core_axis_name="c", subcore_axis_name="s"),
    compiler_params=pltpu.CompilerParams(use_tc_tiling_on_sc=False,
                                         needs_layout_passes=False),
    scratch_types=[
        pltpu.VMEM((DSL, N), jnp.float32),
        pltpu.VMEM((DSL, N), jnp.float32),
        pltpu.VMEM((CHUNK,), jnp.int32),
        pltpu.VMEM((CHUNK,), jnp.int32),
        pltpu.SemaphoreType.DMA,
        pltpu.SemaphoreType.DMA,
    ],
)(_scatter_body)


def kernel(x, edge_index, Wl1, bl1, Wr1, scale1, Wl2, bl2, Wr2, scale2):
    wl1ts = (Wl1 * scale1).T
    wl2ts = (Wl2 * scale2).T
    wr1t = Wr1.T
    wr2t = Wr2.T
    bl1r = bl1.reshape(1, D)
    bl2r = bl2.reshape(1, D)

    pk = _pack_edges(edge_index)
    xnt = _normalize(x)
    pt1 = _sc_scatter(xnt, pk)
    h, hnt = _mid(x, pt1, wl1ts, bl1r, wr1t)
    pt2 = _sc_scatter(hnt, pk)
    return _final(h, pt2, wl2ts, bl2r, wr2t)


# speedup vs baseline: 1.0776x; 1.0590x over previous
import functools

import jax
import jax.numpy as jnp
from jax import lax
from jax.experimental import pallas as pl
from jax.experimental.pallas import tpu as pltpu
from jax.experimental.pallas import tpu_sc as plsc

N = 10000
D = 128
E = 320000
EPS = 1e-12

NWORKERS = 32
DSL = D // NWORKERS

CHUNK = 20000
NCHUNK = E // CHUNK


def _inv_norm(x2):
    return lax.rsqrt(jnp.maximum(x2, EPS * EPS))


def _eye():
    r = lax.broadcasted_iota(jnp.int32, (D, D), 0)
    c = lax.broadcasted_iota(jnp.int32, (D, D), 1)
    return (r == c).astype(jnp.float32)


def _t_out(m):
    return lax.dot_general(_eye(), m, (((1,), (1,)), ((), ())),
                           precision=lax.Precision.HIGHEST,
                           preferred_element_type=jnp.float32)


def _sel(parity):
    r = lax.broadcasted_iota(jnp.int32, (D // 2, D), 0)
    c = lax.broadcasted_iota(jnp.int32, (D // 2, D), 1)
    return (c == 2 * r + parity).astype(jnp.float32)


def _t_pack(m):
    dn = (((1,), (1,)), ((), ()))
    e = lax.dot_general(_sel(0), m, dn, precision=lax.Precision.HIGHEST,
                        preferred_element_type=jnp.float32)
    o = lax.dot_general(_sel(1), m, dn, precision=lax.Precision.HIGHEST,
                        preferred_element_type=jnp.float32)
    ue = lax.bitcast_convert_type(e, jnp.uint32)
    uo = lax.bitcast_convert_type(o, jnp.uint32)
    re = (ue + 0x7FFF + ((ue >> 16) & 1)) >> 16
    ro = (uo + 0x7FFF + ((uo >> 16) & 1)) >> 16
    return lax.bitcast_convert_type(re | (ro << 16), jnp.int32)


def _t_in(mt):
    return lax.dot_general(mt, _eye(), (((0,), (0,)), ((), ())),
                           precision=lax.Precision.HIGHEST,
                           preferred_element_type=jnp.float32)


def _pack_body(ei_ref, pk_ref):
    pk_ref[...] = ei_ref[0:1, :] | (ei_ref[1:2, :] << 16)


_pack_edges = pl.pallas_call(
    _pack_body,
    grid=(10,),
    in_specs=[pl.BlockSpec((2, E // 10), lambda i: (0, i))],
    out_specs=pl.BlockSpec((1, E // 10), lambda i: (0, i)),
    out_shape=jax.ShapeDtypeStruct((1, E), jnp.int32),
)


def _norm_body(x_ref, xnt_ref):
    x = x_ref[...]
    n2 = jnp.sum(x * x, axis=1, keepdims=True)
    xnt_ref[...] = _t_pack(x * _inv_norm(n2))


def _mid_body(x_ref, pt_ref, wlts_ref, bl_ref, wrt_ref, h_ref, hnt_ref):
    x = x_ref[...]
    n2 = jnp.sum(x * x, axis=1, keepdims=True)
    xn = x * _inv_norm(n2)
    agg = xn + _t_in(pt_ref[...])
    a2 = jnp.sum(agg * agg, axis=1, keepdims=True)
    mn = agg * (_inv_norm(a2) * jnp.sqrt(n2))
    out = (jnp.dot(mn, wlts_ref[...], preferred_element_type=jnp.float32)
           + bl_ref[...]
           + jnp.dot(x, wrt_ref[...], preferred_element_type=jnp.float32))
    o2 = jnp.sum(out * out, axis=1, keepdims=True)
    h = jnp.maximum(out * _inv_norm(o2), 0.0)
    h_ref[...] = h
    h2 = jnp.sum(h * h, axis=1, keepdims=True)
    hnt_ref[...] = _t_pack(h * _inv_norm(h2))


def _final_body(x_ref, pt_ref, wlts_ref, bl_ref, wrt_ref, out_ref):
    x = x_ref[...]
    n2 = jnp.sum(x * x, axis=1, keepdims=True)
    xn = x * _inv_norm(n2)
    agg = xn + _t_in(pt_ref[...])
    a2 = jnp.sum(agg * agg, axis=1, keepdims=True)
    mn = agg * (_inv_norm(a2) * jnp.sqrt(n2))
    out = (jnp.dot(mn, wlts_ref[...], preferred_element_type=jnp.float32)
           + bl_ref[...]
           + jnp.dot(x, wrt_ref[...], preferred_element_type=jnp.float32))
    o2 = jnp.sum(out * out, axis=1, keepdims=True)
    out_ref[...] = out * _inv_norm(o2)


_nat = jax.ShapeDtypeStruct((N, D), jnp.float32)
_tr = jax.ShapeDtypeStruct((D, N), jnp.float32)
_trq = jax.ShapeDtypeStruct((D // 2, N), jnp.int32)

_normalize = pl.pallas_call(_norm_body, out_shape=_trq)

_mid = pl.pallas_call(_mid_body, out_shape=[_nat, _trq])

_final = pl.pallas_call(_final_body, out_shape=_nat)


def _scatter_body(xnt_hbm, pk_hbm, out_hbm, xn_t, acc_t, pk_b0, pk_b1, sem0, sem1):
    c = lax.axis_index("c")
    s = lax.axis_index("s")
    wid = s * 2 + c
    d0 = wid * DSL

    first = pltpu.async_copy(pk_hbm.at[0, pl.ds(0, CHUNK)], pk_b0, sem0)
    stage = pltpu.async_copy(xnt_hbm.at[pl.ds(wid * (DSL // 2), DSL // 2), :],
                             xn_t, sem1)

    zeros = jnp.zeros((16,), jnp.float32)

    @plsc.parallel_loop(0, N // 16, unroll=8)
    def _(g):
        for d in range(DSL):
            acc_t[d, pl.ds(g * 16, 16)] = zeros

    stage.wait()

    bufs = [pk_b0, pk_b1]
    sems = [sem0, sem1]
    copies = [first, None]
    for ci in range(NCHUNK):
        if ci + 1 < NCHUNK:
            copies[(ci + 1) % 2] = pltpu.async_copy(
                pk_hbm.at[0, pl.ds((ci + 1) * CHUNK, CHUNK)],
                bufs[(ci + 1) % 2], sems[(ci + 1) % 2])
        copies[ci % 2].wait()
        pk_b = bufs[ci % 2]

        @plsc.parallel_loop(0, CHUNK, step=16, unroll=16)
        def _(b):
            pk_v = pk_b[pl.ds(b, 16)]
            src_v = pk_v & 0xFFFF
            dst_v = pk_v >> 16
            for dp in range(DSL // 2):
                w = plsc.load_gather(xn_t.at[dp], [src_v])
                lo = plsc.bitcast(w << 16, jnp.float32)
                hi = plsc.bitcast(w & jnp.int32(-65536), jnp.float32)
                plsc.addupdate_scatter(acc_t.at[2 * dp], [dst_v], lo)
                plsc.addupdate_scatter(acc_t.at[2 * dp + 1], [dst_v], hi)

    pltpu.sync_copy(acc_t, out_hbm.at[pl.ds(d0, DSL), :])


_sc_scatter = functools.partial(
    pl.kernel,
    out_type=_tr,
    mesh=plsc.VectorSubcoreMesh(core_axis_name="c", subcore_axis_name="s"),
    compiler_params=pltpu.CompilerParams(use_tc_tiling_on_sc=False,
                                         needs_layout_passes=False),
    scratch_types=[
        pltpu.VMEM((DSL // 2, N), jnp.int32),
        pltpu.VMEM((DSL, N), jnp.float32),
        pltpu.VMEM((CHUNK,), jnp.int32),
        pltpu.VMEM((CHUNK,), jnp.int32),
        pltpu.SemaphoreType.DMA,
        pltpu.SemaphoreType.DMA,
    ],
)(_scatter_body)


def kernel(x, edge_index, Wl1, bl1, Wr1, scale1, Wl2, bl2, Wr2, scale2):
    wl1ts = (Wl1 * scale1).T
    wl2ts = (Wl2 * scale2).T
    wr1t = Wr1.T
    wr2t = Wr2.T
    bl1r = bl1.reshape(1, D)
    bl2r = bl2.reshape(1, D)

    pk = _pack_edges(edge_index)
    xnt = _normalize(x)
    pt1 = _sc_scatter(xnt, pk)
    h, hnt = _mid(x, pt1, wl1ts, bl1r, wr1t)
    pt2 = _sc_scatter(hnt, pk)
    return _final(h, pt2, wl2ts, bl2r, wr2t)


# speedup vs baseline: 1.1866x; 1.1012x over previous
import functools

import jax
import jax.numpy as jnp
from jax import lax
from jax.experimental import pallas as pl
from jax.experimental.pallas import tpu as pltpu
from jax.experimental.pallas import tpu_sc as plsc

N = 10000
D = 128
E = 320000
EPS = 1e-12

NWORKERS = 32
DSL = D // NWORKERS

CHUNK = 20000
NCHUNK = E // CHUNK


def _inv_norm(x2):
    return lax.rsqrt(jnp.maximum(x2, EPS * EPS))


def _eye():
    r = lax.broadcasted_iota(jnp.int32, (D, D), 0)
    c = lax.broadcasted_iota(jnp.int32, (D, D), 1)
    return (r == c).astype(jnp.float32)


def _t_out(m):
    return lax.dot_general(_eye(), m, (((1,), (1,)), ((), ())),
                           precision=lax.Precision.HIGHEST,
                           preferred_element_type=jnp.float32)


def _sel(parity):
    r = lax.broadcasted_iota(jnp.int32, (D // 2, D), 0)
    c = lax.broadcasted_iota(jnp.int32, (D // 2, D), 1)
    return (c == 2 * r + parity).astype(jnp.float32)


def _t_pack(m):
    dn = (((1,), (1,)), ((), ()))
    e = lax.dot_general(_sel(0), m, dn, preferred_element_type=jnp.float32)
    o = lax.dot_general(_sel(1), m, dn, preferred_element_type=jnp.float32)
    ue = lax.bitcast_convert_type(e, jnp.uint32)
    uo = lax.bitcast_convert_type(o, jnp.uint32)
    re = (ue + 0x7FFF + ((ue >> 16) & 1)) >> 16
    ro = (uo + 0x7FFF + ((uo >> 16) & 1)) >> 16
    return lax.bitcast_convert_type(re | (ro << 16), jnp.int32)


def _t_in(mt):
    return lax.dot_general(mt, _eye(), (((0,), (0,)), ((), ())),
                           preferred_element_type=jnp.float32)


def _norm_body(x_ref, ei_ref, xnt_ref, pk_ref):
    x = x_ref[...]
    n2 = jnp.sum(x * x, axis=1, keepdims=True)
    xnt_ref[...] = _t_pack(x * _inv_norm(n2))
    pk_ref[...] = ei_ref[0:1, :] | (ei_ref[1:2, :] << 16)


def _mid_body(x_ref, pt_ref, wlts_ref, bl_ref, wrt_ref, h_ref, hnt_ref):
    x = x_ref[...]
    n2 = jnp.sum(x * x, axis=1, keepdims=True)
    xn = x * _inv_norm(n2)
    agg = xn + _t_in(pt_ref[...])
    a2 = jnp.sum(agg * agg, axis=1, keepdims=True)
    mn = agg * (_inv_norm(a2) * jnp.sqrt(n2))
    out = (jnp.dot(mn, wlts_ref[...], preferred_element_type=jnp.float32)
           + bl_ref[...]
           + jnp.dot(x, wrt_ref[...], preferred_element_type=jnp.float32))
    o2 = jnp.sum(out * out, axis=1, keepdims=True)
    h = jnp.maximum(out * _inv_norm(o2), 0.0)
    h_ref[...] = h
    h2 = jnp.sum(h * h, axis=1, keepdims=True)
    hnt_ref[...] = _t_pack(h * _inv_norm(h2))


def _final_body(x_ref, pt_ref, wlts_ref, bl_ref, wrt_ref, out_ref):
    x = x_ref[...]
    n2 = jnp.sum(x * x, axis=1, keepdims=True)
    xn = x * _inv_norm(n2)
    agg = xn + _t_in(pt_ref[...])
    a2 = jnp.sum(agg * agg, axis=1, keepdims=True)
    mn = agg * (_inv_norm(a2) * jnp.sqrt(n2))
    out = (jnp.dot(mn, wlts_ref[...], preferred_element_type=jnp.float32)
           + bl_ref[...]
           + jnp.dot(x, wrt_ref[...], preferred_element_type=jnp.float32))
    o2 = jnp.sum(out * out, axis=1, keepdims=True)
    out_ref[...] = out * _inv_norm(o2)


_nat = jax.ShapeDtypeStruct((N, D), jnp.float32)
_tr = jax.ShapeDtypeStruct((D, N), jnp.float32)
_trq = jax.ShapeDtypeStruct((D // 2, N), jnp.int32)

_normalize = pl.pallas_call(
    _norm_body,
    out_shape=[_trq, jax.ShapeDtypeStruct((1, E), jnp.int32)])

_mid = pl.pallas_call(_mid_body, out_shape=[_nat, _trq])

_final = pl.pallas_call(_final_body, out_shape=_nat)


def _scatter_body(xnt_hbm, pk_hbm, out_hbm, xn_t, acc_t, pk_b0, pk_b1, sem0, sem1):
    c = lax.axis_index("c")
    s = lax.axis_index("s")
    wid = s * 2 + c
    d0 = wid * DSL

    first = pltpu.async_copy(pk_hbm.at[0, pl.ds(0, CHUNK)], pk_b0, sem0)
    stage = pltpu.async_copy(xnt_hbm.at[pl.ds(wid * (DSL // 2), DSL // 2), :],
                             xn_t, sem1)

    zeros = jnp.zeros((16,), jnp.float32)

    @plsc.parallel_loop(0, N // 16, unroll=8)
    def _(g):
        for d in range(DSL):
            acc_t[d, pl.ds(g * 16, 16)] = zeros

    stage.wait()

    bufs = [pk_b0, pk_b1]
    sems = [sem0, sem1]
    copies = [first, None]
    for ci in range(NCHUNK):
        if ci + 1 < NCHUNK:
            copies[(ci + 1) % 2] = pltpu.async_copy(
                pk_hbm.at[0, pl.ds((ci + 1) * CHUNK, CHUNK)],
                bufs[(ci + 1) % 2], sems[(ci + 1) % 2])
        copies[ci % 2].wait()
        pk_b = bufs[ci % 2]

        @plsc.parallel_loop(0, CHUNK, step=16, unroll=16)
        def _(b):
            pk_v = pk_b[pl.ds(b, 16)]
            src_v = pk_v & 0xFFFF
            dst_v = pk_v >> 16
            for dp in range(DSL // 2):
                w = plsc.load_gather(xn_t.at[dp], [src_v])
                lo = plsc.bitcast(w << 16, jnp.float32)
                hi = plsc.bitcast(w & jnp.int32(-65536), jnp.float32)
                plsc.addupdate_scatter(acc_t.at[2 * dp], [dst_v], lo)
                plsc.addupdate_scatter(acc_t.at[2 * dp + 1], [dst_v], hi)

    pltpu.sync_copy(acc_t, out_hbm.at[pl.ds(d0, DSL), :])


_sc_scatter = functools.partial(
    pl.kernel,
    out_type=_tr,
    mesh=plsc.VectorSubcoreMesh(core_axis_name="c", subcore_axis_name="s"),
    compiler_params=pltpu.CompilerParams(use_tc_tiling_on_sc=False,
                                         needs_layout_passes=False),
    scratch_types=[
        pltpu.VMEM((DSL // 2, N), jnp.int32),
        pltpu.VMEM((DSL, N), jnp.float32),
        pltpu.VMEM((CHUNK,), jnp.int32),
        pltpu.VMEM((CHUNK,), jnp.int32),
        pltpu.SemaphoreType.DMA,
        pltpu.SemaphoreType.DMA,
    ],
)(_scatter_body)


def kernel(x, edge_index, Wl1, bl1, Wr1, scale1, Wl2, bl2, Wr2, scale2):
    wl1ts = (Wl1 * scale1).T
    wl2ts = (Wl2 * scale2).T
    wr1t = Wr1.T
    wr2t = Wr2.T
    bl1r = bl1.reshape(1, D)
    bl2r = bl2.reshape(1, D)

    xnt, pk = _normalize(x, edge_index)
    pt1 = _sc_scatter(xnt, pk)
    h, hnt = _mid(x, pt1, wl1ts, bl1r, wr1t)
    pt2 = _sc_scatter(hnt, pk)
    return _final(h, pt2, wl2ts, bl2r, wr2t)


# speedup vs baseline: 1.2160x; 1.0248x over previous
import functools

import jax
import jax.numpy as jnp
from jax import lax
from jax.experimental import pallas as pl
from jax.experimental.pallas import tpu as pltpu
from jax.experimental.pallas import tpu_sc as plsc

N = 10000
D = 128
E = 320000
EPS = 1e-12

NWORKERS = 32
DSL = D // NWORKERS

CHUNK = 20000
NCHUNK = E // CHUNK


def _inv_norm(x2):
    return lax.rsqrt(jnp.maximum(x2, EPS * EPS))


def _eye():
    r = lax.broadcasted_iota(jnp.int32, (D, D), 0)
    c = lax.broadcasted_iota(jnp.int32, (D, D), 1)
    return (r == c).astype(jnp.float32)


def _t_out(m):
    return lax.dot_general(_eye(), m, (((1,), (1,)), ((), ())),
                           precision=lax.Precision.HIGHEST,
                           preferred_element_type=jnp.float32)


def _sel(parity):
    r = lax.broadcasted_iota(jnp.int32, (D // 2, D), 0)
    c = lax.broadcasted_iota(jnp.int32, (D // 2, D), 1)
    return (c == 2 * r + parity).astype(jnp.float32)


def _t_pack(m):
    dn = (((1,), (1,)), ((), ()))
    e = lax.dot_general(_sel(0), m, dn, preferred_element_type=jnp.float32)
    o = lax.dot_general(_sel(1), m, dn, preferred_element_type=jnp.float32)
    ue = lax.bitcast_convert_type(e, jnp.uint32)
    uo = lax.bitcast_convert_type(o, jnp.uint32)
    re = (ue + 0x7FFF + ((ue >> 16) & 1)) >> 16
    ro = (uo + 0x7FFF + ((uo >> 16) & 1)) >> 16
    return lax.bitcast_convert_type(re | (ro << 16), jnp.int32)


def _t_in(mt):
    return lax.dot_general(mt, _eye(), (((0,), (0,)), ((), ())),
                           preferred_element_type=jnp.float32)


def _norm_body(x_ref, ei_ref, xnt_ref, pk_ref):
    x = x_ref[...]
    n2 = jnp.sum(x * x, axis=1, keepdims=True)
    xnt_ref[...] = _t_pack(x * _inv_norm(n2))
    pk_ref[...] = ei_ref[0:1, :] | (ei_ref[1:2, :] << 16)


def _mid_body(x_ref, pt_ref, wlts_ref, bl_ref, wrt_ref, h_ref, hnt_ref):
    x = x_ref[...]
    n2 = jnp.sum(x * x, axis=1, keepdims=True)
    xn = x * _inv_norm(n2)
    agg = xn + _t_in(pt_ref[...])
    a2 = jnp.sum(agg * agg, axis=1, keepdims=True)
    mn = agg * (_inv_norm(a2) * jnp.sqrt(n2))
    out = (jnp.dot(mn, wlts_ref[...], preferred_element_type=jnp.float32)
           + bl_ref[...]
           + jnp.dot(x, wrt_ref[...], preferred_element_type=jnp.float32))
    o2 = jnp.sum(out * out, axis=1, keepdims=True)
    h = jnp.maximum(out * _inv_norm(o2), 0.0)
    h_ref[...] = h
    h2 = jnp.sum(h * h, axis=1, keepdims=True)
    hnt_ref[...] = _t_pack(h * _inv_norm(h2))


def _final_body(x_ref, pt_ref, wlts_ref, bl_ref, wrt_ref, out_ref):
    x = x_ref[...]
    n2 = jnp.sum(x * x, axis=1, keepdims=True)
    xn = x * _inv_norm(n2)
    agg = xn + _t_in(pt_ref[...])
    a2 = jnp.sum(agg * agg, axis=1, keepdims=True)
    mn = agg * (_inv_norm(a2) * jnp.sqrt(n2))
    out = (jnp.dot(mn, wlts_ref[...], preferred_element_type=jnp.float32)
           + bl_ref[...]
           + jnp.dot(x, wrt_ref[...], preferred_element_type=jnp.float32))
    o2 = jnp.sum(out * out, axis=1, keepdims=True)
    out_ref[...] = out * _inv_norm(o2)


_nat = jax.ShapeDtypeStruct((N, D), jnp.float32)
_tr = jax.ShapeDtypeStruct((D, N), jnp.float32)
_trq = jax.ShapeDtypeStruct((D // 2, N), jnp.int32)

_normalize = pl.pallas_call(
    _norm_body,
    out_shape=[_trq, jax.ShapeDtypeStruct((1, E), jnp.int32)])

_mid = pl.pallas_call(_mid_body, out_shape=[_nat, _trq])

_final = pl.pallas_call(_final_body, out_shape=_nat)


def _scatter_body(xnt_hbm, pk_hbm, out_hbm, xn_t, acc_t, pk_b0, pk_b1, sem0, sem1):
    c = lax.axis_index("c")
    s = lax.axis_index("s")
    wid = s * 2 + c
    d0 = wid * DSL

    first = pltpu.async_copy(pk_hbm.at[0, pl.ds(0, CHUNK)], pk_b0, sem0)
    stage = pltpu.async_copy(xnt_hbm.at[pl.ds(wid * (DSL // 2), DSL // 2), :],
                             xn_t, sem1)

    zeros = jnp.zeros((16,), jnp.float32)

    @plsc.parallel_loop(0, N // 16, unroll=8)
    def _(g):
        for d in range(DSL):
            acc_t[d, pl.ds(g * 16, 16)] = zeros

    stage.wait()

    bufs = [pk_b0, pk_b1]
    sems = [sem0, sem1]
    copies = [first, None]
    for ci in range(NCHUNK):
        if ci + 1 < NCHUNK:
            copies[(ci + 1) % 2] = pltpu.async_copy(
                pk_hbm.at[0, pl.ds((ci + 1) * CHUNK, CHUNK)],
                bufs[(ci + 1) % 2], sems[(ci + 1) % 2])
        copies[ci % 2].wait()
        pk_b = bufs[ci % 2]

        @plsc.parallel_loop(0, CHUNK, step=16, unroll=32)
        def _(b):
            pk_v = pk_b[pl.ds(b, 16)]
            src_v = pk_v & 0xFFFF
            dst_v = pk_v >> 16
            for dp in range(DSL // 2):
                w = plsc.load_gather(xn_t.at[dp], [src_v])
                lo = plsc.bitcast(w << 16, jnp.float32)
                hi = plsc.bitcast(w & jnp.int32(-65536), jnp.float32)
                plsc.addupdate_scatter(acc_t.at[2 * dp], [dst_v], lo)
                plsc.addupdate_scatter(acc_t.at[2 * dp + 1], [dst_v], hi)

    pltpu.sync_copy(acc_t, out_hbm.at[pl.ds(d0, DSL), :])


_sc_scatter = functools.partial(
    pl.kernel,
    out_type=_tr,
    mesh=plsc.VectorSubcoreMesh(core_axis_name="c", subcore_axis_name="s"),
    compiler_params=pltpu.CompilerParams(use_tc_tiling_on_sc=False,
                                         needs_layout_passes=False),
    scratch_types=[
        pltpu.VMEM((DSL // 2, N), jnp.int32),
        pltpu.VMEM((DSL, N), jnp.float32),
        pltpu.VMEM((CHUNK,), jnp.int32),
        pltpu.VMEM((CHUNK,), jnp.int32),
        pltpu.SemaphoreType.DMA,
        pltpu.SemaphoreType.DMA,
    ],
)(_scatter_body)


def kernel(x, edge_index, Wl1, bl1, Wr1, scale1, Wl2, bl2, Wr2, scale2):
    wl1ts = (Wl1 * scale1).T
    wl2ts = (Wl2 * scale2).T
    wr1t = Wr1.T
    wr2t = Wr2.T
    bl1r = bl1.reshape(1, D)
    bl2r = bl2.reshape(1, D)

    xnt, pk = _normalize(x, edge_index)
    pt1 = _sc_scatter(xnt, pk)
    h, hnt = _mid(x, pt1, wl1ts, bl1r, wr1t)
    pt2 = _sc_scatter(hnt, pk)
    return _final(h, pt2, wl2ts, bl2r, wr2t)
